# Initial kernel scaffold; baseline (speedup 1.0000x reference)
import jax
import jax.numpy as jnp
from jax.experimental import pallas as pl


def kernel(x):
    raise NotImplementedError("write your pallas kernel here")


# speedup vs baseline: 2.6230x; 2.6230x over previous
import functools

import jax
import jax.numpy as jnp
from jax import lax
from jax.experimental import pallas as pl
from jax.experimental.pallas import tpu as pltpu
from jax.experimental.pallas import tpu_sc as plsc

_EPS = 0.01
_PAD = 4096
_ROW = 4096
_NW = 32
_RPT = _PAD // _NW
_ZW = 32768
_NZB = (_RPT * _ROW) // _ZW


def _toz_body(x_hbm, out_hbm, zbuf, patch, xrow, sem):
    wid = lax.axis_index("s") * 2 + lax.axis_index("c")

    zv = jnp.zeros((16,), jnp.float32)
    lane = lax.iota(jnp.int32, 16)

    def _zero_block(j, carry):
        for u in range(8):
            zbuf[pl.ds((j * 8 + u) * 16, 16)] = zv
        return carry

    lax.fori_loop(0, _ZW // (8 * 16), _zero_block, 0)

    for r in range(16):
        patch[r, :] = jnp.where(lane == r, _EPS, 0.0).astype(jnp.float32)

    row0 = wid * _RPT
    base = (1 + row0) * _ROW
    copies = []
    for b in range(_NZB):
        c = pltpu.make_async_copy(zbuf, out_hbm.at[pl.ds(base + b * _ZW, _ZW)], sem)
        c.start()
        copies.append(c)

    @pl.when(wid == 0)
    def _():
        pltpu.sync_copy(x_hbm, xrow)
        pltpu.sync_copy(xrow, out_hbm.at[pl.ds(0, _ROW)])

    for c in copies:
        c.wait()

    patches = []
    for j in range(_RPT):
        off = base + j * _ROW + (row0 // 16 + j // 16) * 16
        c = pltpu.make_async_copy(patch.at[j % 16], out_hbm.at[pl.ds(off, 16)], sem)
        c.start()
        patches.append(c)
    for c in patches:
        c.wait()


@functools.partial(jax.jit, static_argnums=())
def kernel(x):
    k = pl.kernel(
        _toz_body,
        out_type=jax.ShapeDtypeStruct(((1 + _PAD) * _ROW,), jnp.float32),
        mesh=plsc.VectorSubcoreMesh(core_axis_name="c", subcore_axis_name="s"),
        scratch_types=[
            pltpu.VMEM((_ZW,), jnp.float32),
            pltpu.VMEM((16, 16), jnp.float32),
            pltpu.VMEM((_ROW,), jnp.float32),
            pltpu.SemaphoreType.DMA,
        ],
    )
    flat = k(x.reshape(_ROW))
    return flat.reshape((1 + _PAD,) + x.shape[1:])


# speedup vs baseline: 3.1326x; 1.1943x over previous
import functools

import jax
import jax.numpy as jnp
from jax import lax
from jax.experimental import pallas as pl
from jax.experimental.pallas import tpu as pltpu
from jax.experimental.pallas import tpu_sc as plsc

_EPS = 0.01
_PAD = 4096
_NW = 32
_RPT = _PAD // _NW
_BR = 4
_NB = _RPT // _BR
_NBUF = 2


def _eps_group(b, r):
    col = (_BR * b) % 64 + r
    return (col // 16) * 16, col % 16


def _toz_body(x_hbm, out_hbm, buf0, buf1, xrow, sem0, sem1):
    wid = lax.axis_index("s") * 2 + lax.axis_index("c")
    bufs, sems = [buf0, buf1], [sem0, sem1]

    zv = jnp.zeros((16,), jnp.float32)
    lane = lax.iota(jnp.int32, 16)

    def _zero_block(i, carry):
        r = i // 64
        p = lax.rem(i, 64)
        for u in range(4):
            for buf in bufs:
                buf[r, 0, p, pl.ds(u * 16, 16)] = zv
        return carry

    lax.fori_loop(0, _BR * 64, _zero_block, 0)

    @pl.when(wid == 0)
    def _():
        pltpu.sync_copy(x_hbm, xrow)
        pltpu.sync_copy(xrow, out_hbm.at[pl.ds(0, 1)])

    row0 = wid * _RPT
    copies = [None] * _NB
    for b in range(_NB):
        k = b % _NBUF
        buf = bufs[k]
        plane = 2 * wid + (b * _BR) // 64
        if b >= _NBUF:
            copies[b - _NBUF].wait()
            old_plane = 2 * wid + ((b - _NBUF) * _BR) // 64
            for r in range(_BR):
                g, _ = _eps_group(b - _NBUF, r)
                buf[r, 0, old_plane, pl.ds(g, 16)] = zv
        for r in range(_BR):
            g, l = _eps_group(b, r)
            buf[r, 0, plane, pl.ds(g, 16)] = jnp.where(lane == l, _EPS, 0.0)
        dst = out_hbm.at[pl.ds(1 + row0 + b * _BR, _BR)]
        c = pltpu.make_async_copy(buf, dst, sems[k])
        c.start()
        copies[b] = c
    for b in range(_NB - _NBUF, _NB):
        copies[b].wait()


@functools.partial(jax.jit, static_argnums=())
def kernel(x):
    k = pl.kernel(
        _toz_body,
        out_type=jax.ShapeDtypeStruct((1 + _PAD, 1, 64, 64), jnp.float32),
        mesh=plsc.VectorSubcoreMesh(core_axis_name="c", subcore_axis_name="s"),
        scratch_types=[
            pltpu.VMEM((_BR, 1, 64, 64), jnp.float32),
            pltpu.VMEM((_BR, 1, 64, 64), jnp.float32),
            pltpu.VMEM((1, 1, 64, 64), jnp.float32),
            pltpu.SemaphoreType.DMA,
            pltpu.SemaphoreType.DMA,
        ],
    )
    return k(x)
